# baseline (speedup 1.0000x reference)
import functools

import jax
import jax.numpy as jnp
from jax import lax
from jax.experimental import pallas as pl
from jax.experimental.pallas import tpu as pltpu
from jax.experimental.pallas import tpu_sc as plsc

N, D, H = 256, 128, 512
B_ = 64
NI = N // B_
NP = NI * (NI + 1) // 2
R = B_ * B_
F32 = jnp.float32

NC, NS, L = 2, 16, 16
NW = NC * NS
ROWS_W = N // NW


def _sc_body(i2_ref, mx_ref, mn_ref, bufs_ref, mxbuf_ref, mnbuf_ref, sems):
    wid = lax.axis_index("s") * NC + lax.axis_index("c")
    base = wid * ROWS_W
    copies = [pltpu.async_copy(i2_ref.at[base], bufs_ref.at[0], sems.at[0])]
    for r in range(ROWS_W):
        if r + 1 < ROWS_W:
            copies.append(pltpu.async_copy(i2_ref.at[base + r + 1],
                                           bufs_ref.at[(r + 1) % 2],
                                           sems.at[(r + 1) % 2]))
        copies[r].wait()
        bb = r % 2
        mxs = [bufs_ref[bb, 0, pl.ds(g * L, L)] for g in range(D // L)]
        mns = list(mxs)

        def step(jj, carry):
            cmx, cmn = carry
            vals = [bufs_ref[bb, jj, pl.ds(g * L, L)] for g in range(D // L)]
            return (tuple(jnp.maximum(m, v) for m, v in zip(cmx, vals)),
                    tuple(jnp.minimum(m, v) for m, v in zip(cmn, vals)))

        mxs, mns = lax.fori_loop(1, N, step, (tuple(mxs), tuple(mns)))
        for g in range(D // L):
            mxbuf_ref[pl.ds(g * L, L)] = mxs[g]
            mnbuf_ref[pl.ds(g * L, L)] = mns[g]
        pltpu.sync_copy(mxbuf_ref, mx_ref.at[base + r])
        pltpu.sync_copy(mnbuf_ref, mn_ref.at[base + r])


def _sc_reduce(i2):
    mesh = plsc.VectorSubcoreMesh(core_axis_name="c", subcore_axis_name="s")
    k = functools.partial(
        pl.kernel,
        out_type=[jax.ShapeDtypeStruct((N, D), F32),
                  jax.ShapeDtypeStruct((N, D), F32)],
        mesh=mesh,
        scratch_types=[
            pltpu.VMEM((2, N, D), F32),
            pltpu.VMEM((D,), F32),
            pltpu.VMEM((D,), F32),
            pltpu.SemaphoreType.DMA((2,)),
        ],
    )(_sc_body)
    return k(i2)


def _pair(p):
    i = jnp.int32(0)
    start = 0
    for r in range(1, NI):
        start += NI - (r - 1)
        i = i + (p >= start).astype(jnp.int32)
    base = i * NI - (i * (i - 1)) // 2
    j = p - base + i
    return i, j


def _big_body(bits_ref, i1_ref, x_ref, y_ref,
              w1et_ref, w1eb_ref, b1e_ref, w2e_ref, b2e_ref,
              w1dt_ref, w1db_ref, b1d_ref, w2d_ref, b2d_ref,
              out2_ref, a_ref, bm_ref, st_ref, sem):
    p = pl.program_id(0)
    I, J = _pair(p)
    b = lax.rem(p, 2)
    rows_i = pl.ds(I * B_, B_)
    rows_j = pl.ds(J * B_, B_)

    @pl.when(p == 0)
    def _prep():
        i1v = i1_ref[...]
        a_ref[...] = (jnp.dot(i1v, w1et_ref[...], preferred_element_type=F32)
                      + b1e_ref[...])
        bm_ref[...] = jnp.dot(i1v, w1eb_ref[...], preferred_element_type=F32)

    x = x_ref[...]
    y = y_ref[...]

    bits5 = bits_ref[5]
    bits6 = bits_ref[6]

    def exp_term(arows, brows):
        he = jnp.maximum(a_ref[arows, :][:, None, :]
                         + bm_ref[brows, :][None, :, :], 0.0).reshape(R, H)
        return bits5 * (jnp.dot(he, w2e_ref[...],
                                preferred_element_type=F32) + b2e_ref[...])

    def dir_term(u, v_t):
        hd = jnp.maximum(
            jnp.dot(u.reshape(R, D), w1dt_ref[...],
                    preferred_element_type=F32)
            + jnp.dot(v_t.reshape(R, D), w1db_ref[...],
                      preferred_element_type=F32)
            + b1d_ref[...], 0.0)
        return bits6 * (jnp.dot(hd, w2d_ref[...], preferred_element_type=F32)
                        + b2d_ref[...])

    z0 = exp_term(rows_i, rows_j)
    st_ref[0, b] = jax.nn.sigmoid(
        z0 + dir_term(x, jnp.swapaxes(y, 0, 1))).reshape(B_, B_, D)
    pltpu.make_async_copy(st_ref.at[0, b],
                          out2_ref.at[rows_i, rows_j, :],
                          sem.at[0, b]).start()

    @pl.when(I != J)
    def _():
        z1 = exp_term(rows_j, rows_i)
        st_ref[1, b] = jax.nn.sigmoid(
            z1 + dir_term(y, jnp.swapaxes(x, 0, 1))).reshape(B_, B_, D)
        pltpu.make_async_copy(st_ref.at[1, b],
                              out2_ref.at[rows_j, rows_i, :],
                              sem.at[1, b]).start()

    @pl.when(p > 0)
    def _():
        q = p - 1
        Iq, Jq = _pair(q)
        bq = lax.rem(q, 2)
        qri = pl.ds(Iq * B_, B_)
        qrj = pl.ds(Jq * B_, B_)
        pltpu.make_async_copy(st_ref.at[0, bq],
                              out2_ref.at[qri, qrj, :],
                              sem.at[0, bq]).wait()

        @pl.when(Iq != Jq)
        def _():
            pltpu.make_async_copy(st_ref.at[1, bq],
                                  out2_ref.at[qrj, qri, :],
                                  sem.at[1, bq]).wait()

    @pl.when(p == NP - 1)
    def _tail():
        pltpu.make_async_copy(st_ref.at[0, b],
                              out2_ref.at[rows_i, rows_j, :],
                              sem.at[0, b]).wait()

        @pl.when(I != J)
        def _():
            pltpu.make_async_copy(st_ref.at[1, b],
                                  out2_ref.at[rows_j, rows_i, :],
                                  sem.at[1, b]).wait()


def _mlp(x, w1, b1, w2, b2):
    h = jnp.maximum(jnp.dot(x, w1, preferred_element_type=F32) + b1, 0.0)
    return jnp.dot(h, w2, preferred_element_type=F32) + b2


def _small_body(bits_ref, i0_ref, i1_ref, mx2_ref, mn2_ref,
                w1g0d_ref, b1g0d_ref, w2g0d_ref, b2g0d_ref,
                w1g0rx_ref, w1g0rn_ref, b1g0r_ref, w2g0r_ref, b2g0r_ref,
                w1g1e_ref, b1g1e_ref, w2g1e_ref, b2g1e_ref,
                w1g1d_ref, b1g1d_ref, w2g1d_ref, b2g1d_ref,
                w1g1rx_ref, w1g1rn_ref, b1g1r_ref, w2g1r_ref, b2g1r_ref,
                out1_ref, out0_ref):
    i0v = i0_ref[...]
    i1v = i1_ref[...]
    mx1 = jnp.max(i1v, axis=0, keepdims=True)
    mn1 = jnp.min(i1v, axis=0, keepdims=True)
    o0d = _mlp(i0v, w1g0d_ref[...], b1g0d_ref[...],
               w2g0d_ref[...], b2g0d_ref[...])
    h0r = jnp.maximum(
        jnp.dot(mx1, w1g0rx_ref[...], preferred_element_type=F32)
        + jnp.dot(mn1, w1g0rn_ref[...], preferred_element_type=F32)
        + b1g0r_ref[...], 0.0)
    o0r = jnp.dot(h0r, w2g0r_ref[...], preferred_element_type=F32) \
        + b2g0r_ref[...]
    out0_ref[...] = jax.nn.sigmoid(bits_ref[0] * o0d + bits_ref[1] * o0r)

    e1 = _mlp(i0v, w1g1e_ref[...], b1g1e_ref[...],
              w2g1e_ref[...], b2g1e_ref[...])
    o1d = _mlp(i1v, w1g1d_ref[...], b1g1d_ref[...],
               w2g1d_ref[...], b2g1d_ref[...])
    h1r = jnp.maximum(
        jnp.dot(mx2_ref[...], w1g1rx_ref[...], preferred_element_type=F32)
        + jnp.dot(mn2_ref[...], w1g1rn_ref[...], preferred_element_type=F32)
        + b1g1r_ref[...], 0.0)
    o1r = jnp.dot(h1r, w2g1r_ref[...], preferred_element_type=F32) \
        + b2g1r_ref[...]
    out1_ref[...] = jax.nn.sigmoid(
        bits_ref[2] * e1 + bits_ref[3] * o1d + bits_ref[4] * o1r)


def kernel(inputs_0, inputs_1, inputs_2, action, params):
    action = jnp.asarray(action)
    bits = ((action >> (6 - jnp.arange(7, dtype=action.dtype))) & 1).astype(F32)

    i0 = inputs_0.reshape(1, D)
    i1 = inputs_1.reshape(N, D)
    i2 = inputs_2.reshape(N, N, D)

    def row(p):
        return {k: (v.reshape(1, -1) if v.ndim == 1 else v)
                for k, v in p.items()}

    g2e = row(params['g2_exp'])
    g2d = row(params['g2_dir'])
    g0d = row(params['g0_dir'])
    g0r = row(params['g0_red'])
    g1e = row(params['g1_exp'])
    g1d = row(params['g1_dir'])
    g1r = row(params['g1_red'])

    mx2, mn2 = _sc_reduce(i2)

    const2 = lambda shape: pl.BlockSpec(shape, lambda p: (0, 0))

    def xmap(p):
        i, j = _pair(p)
        return (i, j, 0)

    def ymap(p):
        i, j = _pair(p)
        return (j, i, 0)

    big_in_specs = [
        pl.BlockSpec(memory_space=pltpu.SMEM),
        const2((N, D)),
        pl.BlockSpec((B_, B_, D), xmap),
        pl.BlockSpec((B_, B_, D), ymap),
        const2((D, H)), const2((D, H)), const2((1, H)),
        const2((H, D)), const2((1, D)),
        const2((D, H)), const2((D, H)), const2((1, H)),
        const2((H, D)), const2((1, D)),
    ]
    out2 = pl.pallas_call(
        _big_body,
        grid=(NP,),
        in_specs=big_in_specs,
        out_specs=pl.BlockSpec(memory_space=pl.ANY),
        out_shape=jax.ShapeDtypeStruct((N, N, D), F32),
        scratch_shapes=[
            pltpu.VMEM((N, H), F32),
            pltpu.VMEM((N, H), F32),
            pltpu.VMEM((2, 2, B_, B_, D), F32),
            pltpu.SemaphoreType.DMA((2, 2)),
        ],
        compiler_params=pltpu.CompilerParams(
            dimension_semantics=("arbitrary",)),
    )(
        bits, i1, i2, i2,
        g2e['W1'][:D], g2e['W1'][D:], g2e['b1'], g2e['W2'], g2e['b2'],
        g2d['W1'][:D], g2d['W1'][D:], g2d['b1'], g2d['W2'], g2d['b2'],
    )

    small_in_specs = [
        pl.BlockSpec(memory_space=pltpu.SMEM),
        const2((1, D)),
        const2((N, D)),
        const2((N, D)),
        const2((N, D)),
        const2((D, H)), const2((1, H)), const2((H, D)), const2((1, D)),
        const2((D, H)), const2((D, H)), const2((1, H)),
        const2((H, D)), const2((1, D)),
        const2((D, H)), const2((1, H)), const2((H, D)), const2((1, D)),
        const2((D, H)), const2((1, H)), const2((H, D)), const2((1, D)),
        const2((D, H)), const2((D, H)), const2((1, H)),
        const2((H, D)), const2((1, D)),
    ]
    out1, out0 = pl.pallas_call(
        _small_body,
        grid=(1,),
        in_specs=small_in_specs,
        out_specs=[const2((N, D)), const2((1, D))],
        out_shape=[jax.ShapeDtypeStruct((N, D), F32),
                   jax.ShapeDtypeStruct((1, D), F32)],
    )(
        bits, i0, i1, mx2, mn2,
        g0d['W1'], g0d['b1'], g0d['W2'], g0d['b2'],
        g0r['W1'][0::2], g0r['W1'][1::2], g0r['b1'], g0r['W2'], g0r['b2'],
        g1e['W1'], g1e['b1'], g1e['W2'], g1e['b2'],
        g1d['W1'], g1d['b1'], g1d['W2'], g1d['b2'],
        g1r['W1'][0::2], g1r['W1'][1::2], g1r['b1'], g1r['W2'], g1r['b2'],
    )

    B = inputs_1.shape[0]
    return (out0.reshape(B, D),
            out1.reshape(B, N, D),
            out2.reshape(B, N, N, D))


# speedup vs baseline: 1.0213x; 1.0213x over previous
import functools

import jax
import jax.numpy as jnp
from jax import lax
from jax.experimental import pallas as pl
from jax.experimental.pallas import tpu as pltpu
from jax.experimental.pallas import tpu_sc as plsc

N, D, H = 256, 128, 512
B_ = 64
NI = N // B_
NP = NI * (NI + 1) // 2
R = B_ * B_
F32 = jnp.float32

NC, NS, L = 2, 16, 16
NW = NC * NS
ROWS_W = N // NW


def _sc_body(i2_ref, mx_ref, mn_ref, bufs_ref, mxbuf_ref, mnbuf_ref, sems):
    wid = lax.axis_index("s") * NC + lax.axis_index("c")
    base = wid * ROWS_W
    copies = [pltpu.async_copy(i2_ref.at[base], bufs_ref.at[0], sems.at[0])]
    for r in range(ROWS_W):
        if r + 1 < ROWS_W:
            copies.append(pltpu.async_copy(i2_ref.at[base + r + 1],
                                           bufs_ref.at[(r + 1) % 2],
                                           sems.at[(r + 1) % 2]))
        copies[r].wait()
        bb = r % 2
        mxs = [bufs_ref[bb, 0, pl.ds(g * L, L)] for g in range(D // L)]
        mns = list(mxs)

        def step(jj, carry):
            cmx, cmn = carry
            vals = [bufs_ref[bb, jj, pl.ds(g * L, L)] for g in range(D // L)]
            return (tuple(jnp.maximum(m, v) for m, v in zip(cmx, vals)),
                    tuple(jnp.minimum(m, v) for m, v in zip(cmn, vals)))

        mxs, mns = lax.fori_loop(1, N, step, (tuple(mxs), tuple(mns)))
        for g in range(D // L):
            mxbuf_ref[pl.ds(g * L, L)] = mxs[g]
            mnbuf_ref[pl.ds(g * L, L)] = mns[g]
        pltpu.sync_copy(mxbuf_ref, mx_ref.at[base + r])
        pltpu.sync_copy(mnbuf_ref, mn_ref.at[base + r])


def _sc_reduce(i2):
    mesh = plsc.VectorSubcoreMesh(core_axis_name="c", subcore_axis_name="s")
    k = functools.partial(
        pl.kernel,
        out_type=[jax.ShapeDtypeStruct((N, D), F32),
                  jax.ShapeDtypeStruct((N, D), F32)],
        mesh=mesh,
        scratch_types=[
            pltpu.VMEM((2, N, D), F32),
            pltpu.VMEM((D,), F32),
            pltpu.VMEM((D,), F32),
            pltpu.SemaphoreType.DMA((2,)),
        ],
    )(_sc_body)
    return k(i2)


def _pair(p):
    i = jnp.int32(0)
    start = 0
    for r in range(1, NI):
        start += NI - (r - 1)
        i = i + (p >= start).astype(jnp.int32)
    base = i * NI - (i * (i - 1)) // 2
    j = p - base + i
    return i, j


def _big_body(bits_ref, i1_ref, x_ref, y_ref,
              w1e_ref, b1e_ref, w2e_ref, b2e_ref,
              w1d_ref, b1d_ref, w2d_ref, b2d_ref,
              out2_ref, a_ref, bm_ref, st_ref, sem):
    p = pl.program_id(0)
    I, J = _pair(p)
    b = lax.rem(p, 2)
    rows_i = pl.ds(I * B_, B_)
    rows_j = pl.ds(J * B_, B_)

    @pl.when(p == 0)
    def _prep():
        i1v = i1_ref[...]
        a_ref[...] = (jnp.dot(i1v, w1e_ref[0:D, :],
                              preferred_element_type=F32)
                      + b1e_ref[...]).astype(jnp.bfloat16)
        bm_ref[...] = jnp.dot(i1v, w1e_ref[D:, :],
                              preferred_element_type=F32).astype(jnp.bfloat16)

    x = x_ref[...]
    y = y_ref[...]

    bits5 = bits_ref[5]
    bits6 = bits_ref[6]

    def exp_term(arows, brows):
        he = jnp.maximum(a_ref[arows, :][:, None, :]
                         + bm_ref[brows, :][None, :, :],
                         jnp.bfloat16(0.0)).reshape(R, H)
        return bits5 * (jnp.dot(he, w2e_ref[...],
                                preferred_element_type=F32) + b2e_ref[...])

    def dir_term(u, v_t):
        hd = jnp.maximum(
            jnp.dot(u.reshape(R, D), w1d_ref[0:D, :],
                    preferred_element_type=F32)
            + jnp.dot(v_t.reshape(R, D), w1d_ref[D:, :],
                      preferred_element_type=F32)
            + b1d_ref[...], 0.0)
        return bits6 * (jnp.dot(hd, w2d_ref[...], preferred_element_type=F32)
                        + b2d_ref[...])

    z0 = exp_term(rows_i, rows_j)
    st_ref[0, b] = jax.nn.sigmoid(
        z0 + dir_term(x, jnp.swapaxes(y, 0, 1))).reshape(B_, B_, D)
    pltpu.make_async_copy(st_ref.at[0, b],
                          out2_ref.at[rows_i, rows_j, :],
                          sem.at[0, b]).start()

    @pl.when(I != J)
    def _():
        z1 = exp_term(rows_j, rows_i)
        st_ref[1, b] = jax.nn.sigmoid(
            z1 + dir_term(y, jnp.swapaxes(x, 0, 1))).reshape(B_, B_, D)
        pltpu.make_async_copy(st_ref.at[1, b],
                              out2_ref.at[rows_j, rows_i, :],
                              sem.at[1, b]).start()

    @pl.when(p > 0)
    def _():
        q = p - 1
        Iq, Jq = _pair(q)
        bq = lax.rem(q, 2)
        qri = pl.ds(Iq * B_, B_)
        qrj = pl.ds(Jq * B_, B_)
        pltpu.make_async_copy(st_ref.at[0, bq],
                              out2_ref.at[qri, qrj, :],
                              sem.at[0, bq]).wait()

        @pl.when(Iq != Jq)
        def _():
            pltpu.make_async_copy(st_ref.at[1, bq],
                                  out2_ref.at[qrj, qri, :],
                                  sem.at[1, bq]).wait()

    @pl.when(p == NP - 1)
    def _tail():
        pltpu.make_async_copy(st_ref.at[0, b],
                              out2_ref.at[rows_i, rows_j, :],
                              sem.at[0, b]).wait()

        @pl.when(I != J)
        def _():
            pltpu.make_async_copy(st_ref.at[1, b],
                                  out2_ref.at[rows_j, rows_i, :],
                                  sem.at[1, b]).wait()


def _mlp(x, w1, b1, w2, b2):
    h = jnp.maximum(jnp.dot(x, w1, preferred_element_type=F32) + b1, 0.0)
    return jnp.dot(h, w2, preferred_element_type=F32) + b2


def _small_body(bits_ref, i0_ref, i1_ref, mx2_ref, mn2_ref,
                w1g0d_ref, b1g0d_ref, w2g0d_ref, b2g0d_ref,
                w1g0rx_ref, w1g0rn_ref, b1g0r_ref, w2g0r_ref, b2g0r_ref,
                w1g1e_ref, b1g1e_ref, w2g1e_ref, b2g1e_ref,
                w1g1d_ref, b1g1d_ref, w2g1d_ref, b2g1d_ref,
                w1g1rx_ref, w1g1rn_ref, b1g1r_ref, w2g1r_ref, b2g1r_ref,
                out1_ref, out0_ref):
    i0v = i0_ref[...]
    i1v = i1_ref[...]
    mx1 = jnp.max(i1v, axis=0, keepdims=True)
    mn1 = jnp.min(i1v, axis=0, keepdims=True)
    o0d = _mlp(i0v, w1g0d_ref[...], b1g0d_ref[...],
               w2g0d_ref[...], b2g0d_ref[...])
    h0r = jnp.maximum(
        jnp.dot(mx1, w1g0rx_ref[...], preferred_element_type=F32)
        + jnp.dot(mn1, w1g0rn_ref[...], preferred_element_type=F32)
        + b1g0r_ref[...], 0.0)
    o0r = jnp.dot(h0r, w2g0r_ref[...], preferred_element_type=F32) \
        + b2g0r_ref[...]
    out0_ref[...] = jax.nn.sigmoid(bits_ref[0] * o0d + bits_ref[1] * o0r)

    e1 = _mlp(i0v, w1g1e_ref[...], b1g1e_ref[...],
              w2g1e_ref[...], b2g1e_ref[...])
    o1d = _mlp(i1v, w1g1d_ref[...], b1g1d_ref[...],
               w2g1d_ref[...], b2g1d_ref[...])
    h1r = jnp.maximum(
        jnp.dot(mx2_ref[...], w1g1rx_ref[...], preferred_element_type=F32)
        + jnp.dot(mn2_ref[...], w1g1rn_ref[...], preferred_element_type=F32)
        + b1g1r_ref[...], 0.0)
    o1r = jnp.dot(h1r, w2g1r_ref[...], preferred_element_type=F32) \
        + b2g1r_ref[...]
    out1_ref[...] = jax.nn.sigmoid(
        bits_ref[2] * e1 + bits_ref[3] * o1d + bits_ref[4] * o1r)


def kernel(inputs_0, inputs_1, inputs_2, action, params):
    action = jnp.asarray(action)
    bits = ((action >> (6 - jnp.arange(7, dtype=action.dtype))) & 1).astype(F32)

    i0 = inputs_0.reshape(1, D)
    i1 = inputs_1.reshape(N, D)
    i2 = inputs_2.reshape(N, N, D)

    def row(p):
        return {k: (v.reshape(1, -1) if v.ndim == 1 else v)
                for k, v in p.items()}

    g2e = row(params['g2_exp'])
    g2d = row(params['g2_dir'])
    g0d = row(params['g0_dir'])
    g0r = row(params['g0_red'])
    g1e = row(params['g1_exp'])
    g1d = row(params['g1_dir'])
    g1r = row(params['g1_red'])

    mx2, mn2 = _sc_reduce(i2)

    const2 = lambda shape: pl.BlockSpec(shape, lambda p: (0, 0))

    def xmap(p):
        i, j = _pair(p)
        return (i, j, 0)

    def ymap(p):
        i, j = _pair(p)
        return (j, i, 0)

    big_in_specs = [
        pl.BlockSpec(memory_space=pltpu.SMEM),
        const2((N, D)),
        pl.BlockSpec((B_, B_, D), xmap),
        pl.BlockSpec((B_, B_, D), ymap),
        const2((2 * D, H)), const2((1, H)),
        const2((H, D)), const2((1, D)),
        const2((2 * D, H)), const2((1, H)),
        const2((H, D)), const2((1, D)),
    ]
    out2 = pl.pallas_call(
        _big_body,
        grid=(NP,),
        in_specs=big_in_specs,
        out_specs=pl.BlockSpec(memory_space=pl.ANY),
        out_shape=jax.ShapeDtypeStruct((N, N, D), F32),
        scratch_shapes=[
            pltpu.VMEM((N, H), jnp.bfloat16),
            pltpu.VMEM((N, H), jnp.bfloat16),
            pltpu.VMEM((2, 2, B_, B_, D), F32),
            pltpu.SemaphoreType.DMA((2, 2)),
        ],
        compiler_params=pltpu.CompilerParams(
            dimension_semantics=("arbitrary",)),
    )(
        bits, i1, i2, i2,
        g2e['W1'], g2e['b1'], g2e['W2'].astype(jnp.bfloat16), g2e['b2'],
        g2d['W1'], g2d['b1'], g2d['W2'], g2d['b2'],
    )

    small_in_specs = [
        pl.BlockSpec(memory_space=pltpu.SMEM),
        const2((1, D)),
        const2((N, D)),
        const2((N, D)),
        const2((N, D)),
        const2((D, H)), const2((1, H)), const2((H, D)), const2((1, D)),
        const2((D, H)), const2((D, H)), const2((1, H)),
        const2((H, D)), const2((1, D)),
        const2((D, H)), const2((1, H)), const2((H, D)), const2((1, D)),
        const2((D, H)), const2((1, H)), const2((H, D)), const2((1, D)),
        const2((D, H)), const2((D, H)), const2((1, H)),
        const2((H, D)), const2((1, D)),
    ]
    out1, out0 = pl.pallas_call(
        _small_body,
        grid=(1,),
        in_specs=small_in_specs,
        out_specs=[const2((N, D)), const2((1, D))],
        out_shape=[jax.ShapeDtypeStruct((N, D), F32),
                   jax.ShapeDtypeStruct((1, D), F32)],
    )(
        bits, i0, i1, mx2, mn2,
        g0d['W1'], g0d['b1'], g0d['W2'], g0d['b2'],
        g0r['W1'][0::2], g0r['W1'][1::2], g0r['b1'], g0r['W2'], g0r['b2'],
        g1e['W1'], g1e['b1'], g1e['W2'], g1e['b2'],
        g1d['W1'], g1d['b1'], g1d['W2'], g1d['b2'],
        g1r['W1'][0::2], g1r['W1'][1::2], g1r['b1'], g1r['W2'], g1r['b2'],
    )

    B = inputs_1.shape[0]
    return (out0.reshape(B, D),
            out1.reshape(B, N, D),
            out2.reshape(B, N, N, D))


# speedup vs baseline: 1.0520x; 1.0301x over previous
import jax
import jax.numpy as jnp
from jax import lax
from jax.experimental import pallas as pl
from jax.experimental.pallas import tpu as pltpu

N, D, H = 256, 128, 512
B_ = 64
NI = N // B_
OFF = NI * (NI - 1) // 2
NP = OFF + NI // 2
R = B_ * B_
F32 = jnp.float32
BF16 = jnp.bfloat16


def _offpair(p):
    i = jnp.int32(0)
    start = 0
    for r in range(1, NI - 1):
        start += (NI - 1) - (r - 1)
        i = i + (p >= start).astype(jnp.int32)
    base = i * (NI - 1) - (i * (i - 1)) // 2
    j = p - base + i + 1
    return i, j


def _xy_idx(p):
    i, j = _offpair(p)
    d0 = 2 * (p - OFF)
    d1 = d0 + 1
    off = p < OFF
    xi = jnp.where(off, i, d0)
    xj = jnp.where(off, j, d0)
    yi = jnp.where(off, j, d1)
    yj = jnp.where(off, i, d1)
    return xi, xj, yi, yj


def _mlp(x, w1, b1, w2, b2):
    h = jnp.maximum(jnp.dot(x, w1, preferred_element_type=F32) + b1, 0.0)
    return jnp.dot(h, w2, preferred_element_type=F32) + b2


def _body(bits_ref, i0_ref, i1_ref, x_ref, y_ref,
          w1e_ref, b1e_ref, w2e_ref, b2e_ref,
          w1d_ref, b1d_ref, w2d_ref, b2d_ref,
          w1g0d_ref, b1g0d_ref, w2g0d_ref, b2g0d_ref,
          w1g0rx_ref, w1g0rn_ref, b1g0r_ref, w2g0r_ref, b2g0r_ref,
          w1g1e_ref, b1g1e_ref, w2g1e_ref, b2g1e_ref,
          w1g1d_ref, b1g1d_ref, w2g1d_ref, b2g1d_ref,
          w1g1rx_ref, w1g1rn_ref, b1g1r_ref, w2g1r_ref, b2g1r_ref,
          out2_ref, out1_ref, out0_ref,
          accmx_ref, accmn_ref, a_ref, bm_ref, st_ref, sem,
          tt0_ref, tt1_ref):
    p = pl.program_id(0)
    b = lax.rem(p, 2)
    xi, xj, yi, yj = _xy_idx(p)
    xrows = pl.ds(xi * B_, B_)
    xcols = pl.ds(xj * B_, B_)
    yrows = pl.ds(yi * B_, B_)
    ycols = pl.ds(yj * B_, B_)

    @pl.when(p == 0)
    def _prep():
        i1v = i1_ref[...]
        a_ref[...] = (jnp.dot(i1v, w1e_ref[0:D, :],
                              preferred_element_type=F32)
                      + b1e_ref[...]).astype(BF16)
        bm_ref[...] = jnp.dot(i1v, w1e_ref[D:, :],
                              preferred_element_type=F32).astype(BF16)
        accmx_ref[...] = jnp.full((N, D), -jnp.inf, F32)
        accmn_ref[...] = jnp.full((N, D), jnp.inf, F32)

    x = x_ref[...]
    y = y_ref[...]

    @pl.when(p < OFF)
    def _():
        tt0_ref[...] = jnp.swapaxes(y, 0, 1)
        tt1_ref[...] = jnp.swapaxes(x, 0, 1)

    @pl.when(p >= OFF)
    def _():
        tt0_ref[...] = jnp.swapaxes(x, 0, 1)
        tt1_ref[...] = jnp.swapaxes(y, 0, 1)

    bits5 = bits_ref[5]
    bits6 = bits_ref[6]

    def exp_term(arows, brows):
        he = jnp.maximum(a_ref[arows, :][:, None, :]
                         + bm_ref[brows, :][None, :, :],
                         BF16(0.0)).reshape(R, H)
        return bits5 * (jnp.dot(he, w2e_ref[...],
                                preferred_element_type=F32) + b2e_ref[...])

    def dir_term(u, v_t):
        hd = jnp.maximum(
            jnp.dot(u.reshape(R, D), w1d_ref[0:D, :],
                    preferred_element_type=F32)
            + jnp.dot(v_t.reshape(R, D), w1d_ref[D:, :],
                      preferred_element_type=F32)
            + b1d_ref[...], 0.0)
        return bits6 * (jnp.dot(hd, w2d_ref[...], preferred_element_type=F32)
                        + b2d_ref[...])

    st_ref[0, b] = jax.nn.sigmoid(
        exp_term(xrows, xcols) + dir_term(x, tt0_ref[...])).reshape(B_, B_, D)
    pltpu.make_async_copy(st_ref.at[0, b],
                          out2_ref.at[xrows, xcols, :],
                          sem.at[0, b]).start()

    st_ref[1, b] = jax.nn.sigmoid(
        exp_term(yrows, ycols) + dir_term(y, tt1_ref[...])).reshape(B_, B_, D)
    pltpu.make_async_copy(st_ref.at[1, b],
                          out2_ref.at[yrows, ycols, :],
                          sem.at[1, b]).start()

    accmx_ref[xrows, :] = jnp.maximum(accmx_ref[xrows, :], jnp.max(x, 1))
    accmn_ref[xrows, :] = jnp.minimum(accmn_ref[xrows, :], jnp.min(x, 1))
    accmx_ref[yrows, :] = jnp.maximum(accmx_ref[yrows, :], jnp.max(y, 1))
    accmn_ref[yrows, :] = jnp.minimum(accmn_ref[yrows, :], jnp.min(y, 1))

    @pl.when(p > 0)
    def _():
        q = p - 1
        qxi, qxj, qyi, qyj = _xy_idx(q)
        bq = lax.rem(q, 2)
        pltpu.make_async_copy(st_ref.at[0, bq],
                              out2_ref.at[pl.ds(qxi * B_, B_),
                                          pl.ds(qxj * B_, B_), :],
                              sem.at[0, bq]).wait()
        pltpu.make_async_copy(st_ref.at[1, bq],
                              out2_ref.at[pl.ds(qyi * B_, B_),
                                          pl.ds(qyj * B_, B_), :],
                              sem.at[1, bq]).wait()

    @pl.when(p == NP - 1)
    def _tail():
        pltpu.make_async_copy(st_ref.at[0, b],
                              out2_ref.at[xrows, xcols, :],
                              sem.at[0, b]).wait()
        pltpu.make_async_copy(st_ref.at[1, b],
                              out2_ref.at[yrows, ycols, :],
                              sem.at[1, b]).wait()

        i0v = i0_ref[...]
        i1v = i1_ref[...]
        mx1 = jnp.max(i1v, axis=0, keepdims=True)
        mn1 = jnp.min(i1v, axis=0, keepdims=True)
        o0d = _mlp(i0v, w1g0d_ref[...], b1g0d_ref[...],
                   w2g0d_ref[...], b2g0d_ref[...])
        h0r = jnp.maximum(
            jnp.dot(mx1, w1g0rx_ref[...], preferred_element_type=F32)
            + jnp.dot(mn1, w1g0rn_ref[...], preferred_element_type=F32)
            + b1g0r_ref[...], 0.0)
        o0r = jnp.dot(h0r, w2g0r_ref[...], preferred_element_type=F32) \
            + b2g0r_ref[...]
        out0_ref[...] = jax.nn.sigmoid(bits_ref[0] * o0d + bits_ref[1] * o0r)

        e1 = _mlp(i0v, w1g1e_ref[...], b1g1e_ref[...],
                  w2g1e_ref[...], b2g1e_ref[...])
        o1d = _mlp(i1v, w1g1d_ref[...], b1g1d_ref[...],
                   w2g1d_ref[...], b2g1d_ref[...])
        h1r = jnp.maximum(
            jnp.dot(accmx_ref[...], w1g1rx_ref[...],
                    preferred_element_type=F32)
            + jnp.dot(accmn_ref[...], w1g1rn_ref[...],
                      preferred_element_type=F32)
            + b1g1r_ref[...], 0.0)
        o1r = jnp.dot(h1r, w2g1r_ref[...], preferred_element_type=F32) \
            + b2g1r_ref[...]
        out1_ref[...] = jax.nn.sigmoid(
            bits_ref[2] * e1 + bits_ref[3] * o1d + bits_ref[4] * o1r)


def kernel(inputs_0, inputs_1, inputs_2, action, params):
    action = jnp.asarray(action)
    bits = ((action >> (6 - jnp.arange(7, dtype=action.dtype))) & 1).astype(F32)

    i0 = inputs_0.reshape(1, D)
    i1 = inputs_1.reshape(N, D)
    i2 = inputs_2.reshape(N, N, D)

    def row(p):
        return {k: (v.reshape(1, -1) if v.ndim == 1 else v)
                for k, v in p.items()}

    g2e = row(params['g2_exp'])
    g2d = row(params['g2_dir'])
    g0d = row(params['g0_dir'])
    g0r = row(params['g0_red'])
    g1e = row(params['g1_exp'])
    g1d = row(params['g1_dir'])
    g1r = row(params['g1_red'])

    const2 = lambda shape: pl.BlockSpec(shape, lambda p: (0, 0))

    def xmap(p):
        xi, xj, _, _ = _xy_idx(p)
        return (xi, xj, 0)

    def ymap(p):
        _, _, yi, yj = _xy_idx(p)
        return (yi, yj, 0)

    in_specs = [
        pl.BlockSpec(memory_space=pltpu.SMEM),
        const2((1, D)),
        const2((N, D)),
        pl.BlockSpec((B_, B_, D), xmap),
        pl.BlockSpec((B_, B_, D), ymap),
        const2((2 * D, H)), const2((1, H)),
        const2((H, D)), const2((1, D)),
        const2((2 * D, H)), const2((1, H)),
        const2((H, D)), const2((1, D)),
        const2((D, H)), const2((1, H)), const2((H, D)), const2((1, D)),
        const2((D, H)), const2((D, H)), const2((1, H)),
        const2((H, D)), const2((1, D)),
        const2((D, H)), const2((1, H)), const2((H, D)), const2((1, D)),
        const2((D, H)), const2((1, H)), const2((H, D)), const2((1, D)),
        const2((D, H)), const2((D, H)), const2((1, H)),
        const2((H, D)), const2((1, D)),
    ]
    out_specs = [
        pl.BlockSpec(memory_space=pl.ANY),
        const2((N, D)),
        const2((1, D)),
    ]
    out_shape = [
        jax.ShapeDtypeStruct((N, N, D), F32),
        jax.ShapeDtypeStruct((N, D), F32),
        jax.ShapeDtypeStruct((1, D), F32),
    ]
    scratch = [
        pltpu.VMEM((N, D), F32),
        pltpu.VMEM((N, D), F32),
        pltpu.VMEM((N, H), BF16),
        pltpu.VMEM((N, H), BF16),
        pltpu.VMEM((2, 2, B_, B_, D), F32),
        pltpu.SemaphoreType.DMA((2, 2)),
        pltpu.VMEM((B_, B_, D), F32),
        pltpu.VMEM((B_, B_, D), F32),
    ]

    out2, out1, out0 = pl.pallas_call(
        _body,
        grid=(NP,),
        in_specs=in_specs,
        out_specs=out_specs,
        out_shape=out_shape,
        scratch_shapes=scratch,
        compiler_params=pltpu.CompilerParams(
            dimension_semantics=("arbitrary",)),
    )(
        bits, i0, i1, i2, i2,
        g2e['W1'], g2e['b1'], g2e['W2'].astype(BF16), g2e['b2'],
        g2d['W1'], g2d['b1'], g2d['W2'], g2d['b2'],
        g0d['W1'], g0d['b1'], g0d['W2'], g0d['b2'],
        g0r['W1'][0::2], g0r['W1'][1::2], g0r['b1'], g0r['W2'], g0r['b2'],
        g1e['W1'], g1e['b1'], g1e['W2'], g1e['b2'],
        g1d['W1'], g1d['b1'], g1d['W2'], g1d['b2'],
        g1r['W1'][0::2], g1r['W1'][1::2], g1r['b1'], g1r['W2'], g1r['b2'],
    )

    B = inputs_1.shape[0]
    return (out0.reshape(B, D),
            out1.reshape(B, N, D),
            out2.reshape(B, N, N, D))


# speedup vs baseline: 1.1220x; 1.0666x over previous
import jax
import jax.numpy as jnp
from jax import lax
from jax.experimental import pallas as pl
from jax.experimental.pallas import tpu as pltpu

N, D, H = 256, 128, 512
B_ = 64
NI = N // B_
NP = NI * (NI + 1) // 2
R = B_ * B_
F32 = jnp.float32


def _pair(p):
    i = jnp.int32(0)
    start = 0
    for r in range(1, NI):
        start += NI - (r - 1)
        i = i + (p >= start).astype(jnp.int32)
    base = i * NI - (i * (i - 1)) // 2
    j = p - base + i
    return i, j


def _mlp(x, w1, b1, w2, b2):
    h = jnp.maximum(jnp.dot(x, w1, preferred_element_type=F32) + b1, 0.0)
    return jnp.dot(h, w2, preferred_element_type=F32) + b2


def _body(bits_ref, i0_ref, i1_ref, x_ref, y_ref,
          w1e_ref, b1e_ref, w2e_ref, b2e_ref,
          w1d_ref, b1d_ref, w2d_ref, b2d_ref,
          w1g0d_ref, b1g0d_ref, w2g0d_ref, b2g0d_ref,
          w1g0rx_ref, w1g0rn_ref, b1g0r_ref, w2g0r_ref, b2g0r_ref,
          w1g1e_ref, b1g1e_ref, w2g1e_ref, b2g1e_ref,
          w1g1d_ref, b1g1d_ref, w2g1d_ref, b2g1d_ref,
          w1g1rx_ref, w1g1rn_ref, b1g1r_ref, w2g1r_ref, b2g1r_ref,
          out2_ref, out1_ref, out0_ref,
          accmx_ref, accmn_ref, a_ref, bm_ref, st_ref, sem):
    p = pl.program_id(0)
    I, J = _pair(p)
    b = lax.rem(p, 2)
    rows_i = pl.ds(I * B_, B_)
    rows_j = pl.ds(J * B_, B_)

    @pl.when(p == 0)
    def _prep():
        i1v = i1_ref[...]
        a_ref[...] = (jnp.dot(i1v, w1e_ref[0:D, :],
                              preferred_element_type=F32)
                      + b1e_ref[...]).astype(jnp.bfloat16)
        bm_ref[...] = jnp.dot(i1v, w1e_ref[D:, :],
                              preferred_element_type=F32).astype(jnp.bfloat16)
        accmx_ref[...] = jnp.full((N, D), -jnp.inf, F32)
        accmn_ref[...] = jnp.full((N, D), jnp.inf, F32)

    x = x_ref[...]
    y = y_ref[...]

    bits5 = bits_ref[5]
    bits6 = bits_ref[6]

    def exp_term(arows, brows):
        he = jnp.maximum(a_ref[arows, :][:, None, :]
                         + bm_ref[brows, :][None, :, :],
                         jnp.bfloat16(0.0)).reshape(R, H)
        return bits5 * (jnp.dot(he, w2e_ref[...],
                                preferred_element_type=F32) + b2e_ref[...])

    def dir_term(u, v_t):
        hd = jnp.maximum(
            jnp.dot(u.reshape(R, D), w1d_ref[0:D, :],
                    preferred_element_type=F32)
            + jnp.dot(v_t.reshape(R, D), w1d_ref[D:, :],
                      preferred_element_type=F32)
            + b1d_ref[...], 0.0)
        return bits6 * (jnp.dot(hd, w2d_ref[...], preferred_element_type=F32)
                        + b2d_ref[...])

    z0 = exp_term(rows_i, rows_j)
    st_ref[0, b] = jax.nn.sigmoid(
        z0 + dir_term(x, jnp.swapaxes(y, 0, 1))).reshape(B_, B_, D)
    pltpu.make_async_copy(st_ref.at[0, b],
                          out2_ref.at[rows_i, rows_j, :],
                          sem.at[0, b]).start()

    @pl.when(I != J)
    def _():
        z1 = exp_term(rows_j, rows_i)
        st_ref[1, b] = jax.nn.sigmoid(
            z1 + dir_term(y, jnp.swapaxes(x, 0, 1))).reshape(B_, B_, D)
        pltpu.make_async_copy(st_ref.at[1, b],
                              out2_ref.at[rows_j, rows_i, :],
                              sem.at[1, b]).start()

    accmx_ref[rows_i, :] = jnp.maximum(accmx_ref[rows_i, :], jnp.max(x, 1))
    accmn_ref[rows_i, :] = jnp.minimum(accmn_ref[rows_i, :], jnp.min(x, 1))

    @pl.when(I != J)
    def _():
        accmx_ref[rows_j, :] = jnp.maximum(accmx_ref[rows_j, :],
                                           jnp.max(y, 1))
        accmn_ref[rows_j, :] = jnp.minimum(accmn_ref[rows_j, :],
                                           jnp.min(y, 1))

    @pl.when(p > 0)
    def _():
        q = p - 1
        Iq, Jq = _pair(q)
        bq = lax.rem(q, 2)
        qri = pl.ds(Iq * B_, B_)
        qrj = pl.ds(Jq * B_, B_)
        pltpu.make_async_copy(st_ref.at[0, bq],
                              out2_ref.at[qri, qrj, :],
                              sem.at[0, bq]).wait()

        @pl.when(Iq != Jq)
        def _():
            pltpu.make_async_copy(st_ref.at[1, bq],
                                  out2_ref.at[qrj, qri, :],
                                  sem.at[1, bq]).wait()

    @pl.when(p == NP - 1)
    def _tail():
        pltpu.make_async_copy(st_ref.at[0, b],
                              out2_ref.at[rows_i, rows_j, :],
                              sem.at[0, b]).wait()

        @pl.when(I != J)
        def _():
            pltpu.make_async_copy(st_ref.at[1, b],
                                  out2_ref.at[rows_j, rows_i, :],
                                  sem.at[1, b]).wait()

        i0v = i0_ref[...]
        i1v = i1_ref[...]
        mx1 = jnp.max(i1v, axis=0, keepdims=True)
        mn1 = jnp.min(i1v, axis=0, keepdims=True)
        o0d = _mlp(i0v, w1g0d_ref[...], b1g0d_ref[...],
                   w2g0d_ref[...], b2g0d_ref[...])
        h0r = jnp.maximum(
            jnp.dot(mx1, w1g0rx_ref[...], preferred_element_type=F32)
            + jnp.dot(mn1, w1g0rn_ref[...], preferred_element_type=F32)
            + b1g0r_ref[...], 0.0)
        o0r = jnp.dot(h0r, w2g0r_ref[...], preferred_element_type=F32) \
            + b2g0r_ref[...]
        out0_ref[...] = jax.nn.sigmoid(bits_ref[0] * o0d + bits_ref[1] * o0r)

        e1 = _mlp(i0v, w1g1e_ref[...], b1g1e_ref[...],
                  w2g1e_ref[...], b2g1e_ref[...])
        o1d = _mlp(i1v, w1g1d_ref[...], b1g1d_ref[...],
                   w2g1d_ref[...], b2g1d_ref[...])
        h1r = jnp.maximum(
            jnp.dot(accmx_ref[...], w1g1rx_ref[...],
                    preferred_element_type=F32)
            + jnp.dot(accmn_ref[...], w1g1rn_ref[...],
                      preferred_element_type=F32)
            + b1g1r_ref[...], 0.0)
        o1r = jnp.dot(h1r, w2g1r_ref[...], preferred_element_type=F32) \
            + b2g1r_ref[...]
        out1_ref[...] = jax.nn.sigmoid(
            bits_ref[2] * e1 + bits_ref[3] * o1d + bits_ref[4] * o1r)


def kernel(inputs_0, inputs_1, inputs_2, action, params):
    action = jnp.asarray(action)
    bits = ((action >> (6 - jnp.arange(7, dtype=action.dtype))) & 1).astype(F32)

    i0 = inputs_0.reshape(1, D)
    i1 = inputs_1.reshape(N, D)
    i2 = inputs_2.reshape(N, N, D)

    def row(p):
        return {k: (v.reshape(1, -1) if v.ndim == 1 else v)
                for k, v in p.items()}

    g2e = row(params['g2_exp'])
    g2d = row(params['g2_dir'])
    g0d = row(params['g0_dir'])
    g0r = row(params['g0_red'])
    g1e = row(params['g1_exp'])
    g1d = row(params['g1_dir'])
    g1r = row(params['g1_red'])

    const2 = lambda shape: pl.BlockSpec(shape, lambda p: (0, 0))

    def xmap(p):
        i, j = _pair(p)
        return (i, j, 0)

    def ymap(p):
        i, j = _pair(p)
        return (j, i, 0)

    in_specs = [
        pl.BlockSpec(memory_space=pltpu.SMEM),
        const2((1, D)),
        const2((N, D)),
        pl.BlockSpec((B_, B_, D), xmap),
        pl.BlockSpec((B_, B_, D), ymap),
        const2((2 * D, H)), const2((1, H)),
        const2((H, D)), const2((1, D)),
        const2((2 * D, H)), const2((1, H)),
        const2((H, D)), const2((1, D)),
        const2((D, H)), const2((1, H)), const2((H, D)), const2((1, D)),
        const2((D, H)), const2((D, H)), const2((1, H)),
        const2((H, D)), const2((1, D)),
        const2((D, H)), const2((1, H)), const2((H, D)), const2((1, D)),
        const2((D, H)), const2((1, H)), const2((H, D)), const2((1, D)),
        const2((D, H)), const2((D, H)), const2((1, H)),
        const2((H, D)), const2((1, D)),
    ]
    out_specs = [
        pl.BlockSpec(memory_space=pl.ANY),
        const2((N, D)),
        const2((1, D)),
    ]
    out_shape = [
        jax.ShapeDtypeStruct((N, N, D), F32),
        jax.ShapeDtypeStruct((N, D), F32),
        jax.ShapeDtypeStruct((1, D), F32),
    ]
    scratch = [
        pltpu.VMEM((N, D), F32),
        pltpu.VMEM((N, D), F32),
        pltpu.VMEM((N, H), jnp.bfloat16),
        pltpu.VMEM((N, H), jnp.bfloat16),
        pltpu.VMEM((2, 2, B_, B_, D), F32),
        pltpu.SemaphoreType.DMA((2, 2)),
    ]

    out2, out1, out0 = pl.pallas_call(
        _body,
        grid=(NP,),
        in_specs=in_specs,
        out_specs=out_specs,
        out_shape=out_shape,
        scratch_shapes=scratch,
        compiler_params=pltpu.CompilerParams(
            dimension_semantics=("arbitrary",)),
    )(
        bits, i0, i1, i2, i2,
        g2e['W1'], g2e['b1'], g2e['W2'].astype(jnp.bfloat16), g2e['b2'],
        g2d['W1'], g2d['b1'], g2d['W2'], g2d['b2'],
        g0d['W1'], g0d['b1'], g0d['W2'], g0d['b2'],
        g0r['W1'][0::2], g0r['W1'][1::2], g0r['b1'], g0r['W2'], g0r['b2'],
        g1e['W1'], g1e['b1'], g1e['W2'], g1e['b2'],
        g1d['W1'], g1d['b1'], g1d['W2'], g1d['b2'],
        g1r['W1'][0::2], g1r['W1'][1::2], g1r['b1'], g1r['W2'], g1r['b2'],
    )

    B = inputs_1.shape[0]
    return (out0.reshape(B, D),
            out1.reshape(B, N, D),
            out2.reshape(B, N, N, D))


# speedup vs baseline: 1.2697x; 1.1316x over previous
import jax
import jax.numpy as jnp
from jax import lax
from jax.experimental import pallas as pl
from jax.experimental.pallas import tpu as pltpu

N, D, H = 256, 128, 512
B_ = 64
NI = N // B_
NP = NI * (NI + 1) // 2
R = B_ * B_
F32 = jnp.float32


def _pair(p):
    i = jnp.int32(0)
    start = 0
    for r in range(1, NI):
        start += NI - (r - 1)
        i = i + (p >= start).astype(jnp.int32)
    base = i * NI - (i * (i - 1)) // 2
    j = p - base + i
    return i, j


def _mlp(x, w1, b1, w2, b2):
    h = jnp.maximum(jnp.dot(x, w1, preferred_element_type=F32) + b1, 0.0)
    return jnp.dot(h, w2, preferred_element_type=F32) + b2


def _body(bits_ref, i0_ref, i1_ref, x_ref, y_ref,
          w1e_ref, b1e_ref, w2e_ref, b2e_ref,
          w1d_ref, b1d_ref, w2d_ref, b2d_ref,
          w1g0d_ref, b1g0d_ref, w2g0d_ref, b2g0d_ref,
          w1g0r_ref, b1g0r_ref, w2g0r_ref, b2g0r_ref,
          w1g1e_ref, b1g1e_ref, w2g1e_ref, b2g1e_ref,
          w1g1d_ref, b1g1d_ref, w2g1d_ref, b2g1d_ref,
          w1g1r_ref, b1g1r_ref, w2g1r_ref, b2g1r_ref,
          out2_ref, out1_ref, out0_ref,
          accmx_ref, accmn_ref, a_ref, bm_ref, st_ref, sem, w2eb_ref,
          w0x_ref, w0n_ref, w1x_ref, w1n_ref):
    p = pl.program_id(0)
    I, J = _pair(p)
    b = lax.rem(p, 2)
    rows_i = pl.ds(I * B_, B_)
    rows_j = pl.ds(J * B_, B_)

    @pl.when(p == 0)
    def _prep():
        i1v = i1_ref[...]
        a_ref[...] = (jnp.dot(i1v, w1e_ref[0:D, :],
                              preferred_element_type=F32)
                      + b1e_ref[...]).astype(jnp.bfloat16)
        bm_ref[...] = jnp.dot(i1v, w1e_ref[D:, :],
                              preferred_element_type=F32).astype(jnp.bfloat16)
        w2eb_ref[...] = w2e_ref[...].astype(jnp.bfloat16)
        accmx_ref[...] = jnp.full((N, D), -jnp.inf, F32)
        accmn_ref[...] = jnp.full((N, D), jnp.inf, F32)
        ii = lax.broadcasted_iota(jnp.int32, (D, 2 * D), 0)
        jj = lax.broadcasted_iota(jnp.int32, (D, 2 * D), 1)
        se = (jj == 2 * ii).astype(F32)
        so = (jj == 2 * ii + 1).astype(F32)
        w0x_ref[...] = jnp.dot(se, w1g0r_ref[...], preferred_element_type=F32)
        w0n_ref[...] = jnp.dot(so, w1g0r_ref[...], preferred_element_type=F32)
        w1x_ref[...] = jnp.dot(se, w1g1r_ref[...], preferred_element_type=F32)
        w1n_ref[...] = jnp.dot(so, w1g1r_ref[...], preferred_element_type=F32)

    x = x_ref[...]
    y = y_ref[...]

    bits5 = bits_ref[5]
    bits6 = bits_ref[6]

    def exp_term(arows, brows):
        he = jnp.maximum(a_ref[arows, :][:, None, :]
                         + bm_ref[brows, :][None, :, :],
                         jnp.bfloat16(0.0)).reshape(R, H)
        return bits5 * (jnp.dot(he, w2eb_ref[...],
                                preferred_element_type=F32) + b2e_ref[...])

    def dir_term(u, v_t):
        hd = jnp.maximum(
            jnp.dot(u.reshape(R, D), w1d_ref[0:D, :],
                    preferred_element_type=F32)
            + jnp.dot(v_t.reshape(R, D), w1d_ref[D:, :],
                      preferred_element_type=F32)
            + b1d_ref[...], 0.0)
        return bits6 * (jnp.dot(hd, w2d_ref[...], preferred_element_type=F32)
                        + b2d_ref[...])

    z0 = exp_term(rows_i, rows_j)
    st_ref[0, b] = jax.nn.sigmoid(
        z0 + dir_term(x, jnp.swapaxes(y, 0, 1))).reshape(B_, B_, D)
    pltpu.make_async_copy(st_ref.at[0, b],
                          out2_ref.at[rows_i, rows_j, :],
                          sem.at[0, b]).start()

    @pl.when(I != J)
    def _():
        z1 = exp_term(rows_j, rows_i)
        st_ref[1, b] = jax.nn.sigmoid(
            z1 + dir_term(y, jnp.swapaxes(x, 0, 1))).reshape(B_, B_, D)
        pltpu.make_async_copy(st_ref.at[1, b],
                              out2_ref.at[rows_j, rows_i, :],
                              sem.at[1, b]).start()

    accmx_ref[rows_i, :] = jnp.maximum(accmx_ref[rows_i, :], jnp.max(x, 1))
    accmn_ref[rows_i, :] = jnp.minimum(accmn_ref[rows_i, :], jnp.min(x, 1))

    @pl.when(I != J)
    def _():
        accmx_ref[rows_j, :] = jnp.maximum(accmx_ref[rows_j, :],
                                           jnp.max(y, 1))
        accmn_ref[rows_j, :] = jnp.minimum(accmn_ref[rows_j, :],
                                           jnp.min(y, 1))

    @pl.when(p > 0)
    def _():
        q = p - 1
        Iq, Jq = _pair(q)
        bq = lax.rem(q, 2)
        qri = pl.ds(Iq * B_, B_)
        qrj = pl.ds(Jq * B_, B_)
        pltpu.make_async_copy(st_ref.at[0, bq],
                              out2_ref.at[qri, qrj, :],
                              sem.at[0, bq]).wait()

        @pl.when(Iq != Jq)
        def _():
            pltpu.make_async_copy(st_ref.at[1, bq],
                                  out2_ref.at[qrj, qri, :],
                                  sem.at[1, bq]).wait()

    @pl.when(p == NP - 1)
    def _tail():
        pltpu.make_async_copy(st_ref.at[0, b],
                              out2_ref.at[rows_i, rows_j, :],
                              sem.at[0, b]).wait()

        @pl.when(I != J)
        def _():
            pltpu.make_async_copy(st_ref.at[1, b],
                                  out2_ref.at[rows_j, rows_i, :],
                                  sem.at[1, b]).wait()

        i0v = i0_ref[...]
        i1v = i1_ref[...]
        mx1 = jnp.max(i1v, axis=0, keepdims=True)
        mn1 = jnp.min(i1v, axis=0, keepdims=True)
        o0d = _mlp(i0v, w1g0d_ref[...], b1g0d_ref[...],
                   w2g0d_ref[...], b2g0d_ref[...])
        h0r = jnp.maximum(
            jnp.dot(mx1, w0x_ref[...], preferred_element_type=F32)
            + jnp.dot(mn1, w0n_ref[...], preferred_element_type=F32)
            + b1g0r_ref[...], 0.0)
        o0r = jnp.dot(h0r, w2g0r_ref[...], preferred_element_type=F32) \
            + b2g0r_ref[...]
        out0_ref[...] = jax.nn.sigmoid(bits_ref[0] * o0d + bits_ref[1] * o0r)

        e1 = _mlp(i0v, w1g1e_ref[...], b1g1e_ref[...],
                  w2g1e_ref[...], b2g1e_ref[...])
        o1d = _mlp(i1v, w1g1d_ref[...], b1g1d_ref[...],
                   w2g1d_ref[...], b2g1d_ref[...])
        h1r = jnp.maximum(
            jnp.dot(accmx_ref[...], w1x_ref[...], preferred_element_type=F32)
            + jnp.dot(accmn_ref[...], w1n_ref[...],
                      preferred_element_type=F32)
            + b1g1r_ref[...], 0.0)
        o1r = jnp.dot(h1r, w2g1r_ref[...], preferred_element_type=F32) \
            + b2g1r_ref[...]
        out1_ref[...] = jax.nn.sigmoid(
            bits_ref[2] * e1 + bits_ref[3] * o1d + bits_ref[4] * o1r)


def kernel(inputs_0, inputs_1, inputs_2, action, params):
    action = jnp.asarray(action)
    bits = ((action >> (6 - jnp.arange(7, dtype=action.dtype))) & 1).astype(F32)

    i0 = inputs_0.reshape(1, D)
    i1 = inputs_1.reshape(N, D)
    i2 = inputs_2.reshape(N, N, D)

    def row(p):
        return {k: (v.reshape(1, -1) if v.ndim == 1 else v)
                for k, v in p.items()}

    g2e = row(params['g2_exp'])
    g2d = row(params['g2_dir'])
    g0d = row(params['g0_dir'])
    g0r = row(params['g0_red'])
    g1e = row(params['g1_exp'])
    g1d = row(params['g1_dir'])
    g1r = row(params['g1_red'])

    const2 = lambda shape: pl.BlockSpec(shape, lambda p: (0, 0))

    def xmap(p):
        i, j = _pair(p)
        return (i, j, 0)

    def ymap(p):
        i, j = _pair(p)
        return (j, i, 0)

    in_specs = [
        pl.BlockSpec(memory_space=pltpu.SMEM),
        const2((1, D)),
        const2((N, D)),
        pl.BlockSpec((B_, B_, D), xmap),
        pl.BlockSpec((B_, B_, D), ymap),
        const2((2 * D, H)), const2((1, H)),
        const2((H, D)), const2((1, D)),
        const2((2 * D, H)), const2((1, H)),
        const2((H, D)), const2((1, D)),
        const2((D, H)), const2((1, H)), const2((H, D)), const2((1, D)),
        const2((2 * D, H)), const2((1, H)),
        const2((H, D)), const2((1, D)),
        const2((D, H)), const2((1, H)), const2((H, D)), const2((1, D)),
        const2((D, H)), const2((1, H)), const2((H, D)), const2((1, D)),
        const2((2 * D, H)), const2((1, H)),
        const2((H, D)), const2((1, D)),
    ]
    out_specs = [
        pl.BlockSpec(memory_space=pl.ANY),
        const2((N, D)),
        const2((1, D)),
    ]
    out_shape = [
        jax.ShapeDtypeStruct((N, N, D), F32),
        jax.ShapeDtypeStruct((N, D), F32),
        jax.ShapeDtypeStruct((1, D), F32),
    ]
    scratch = [
        pltpu.VMEM((N, D), F32),
        pltpu.VMEM((N, D), F32),
        pltpu.VMEM((N, H), jnp.bfloat16),
        pltpu.VMEM((N, H), jnp.bfloat16),
        pltpu.VMEM((2, 2, B_, B_, D), F32),
        pltpu.SemaphoreType.DMA((2, 2)),
        pltpu.VMEM((H, D), jnp.bfloat16),
        pltpu.VMEM((D, H), F32),
        pltpu.VMEM((D, H), F32),
        pltpu.VMEM((D, H), F32),
        pltpu.VMEM((D, H), F32),
    ]

    out2, out1, out0 = pl.pallas_call(
        _body,
        grid=(NP,),
        in_specs=in_specs,
        out_specs=out_specs,
        out_shape=out_shape,
        scratch_shapes=scratch,
        compiler_params=pltpu.CompilerParams(
            dimension_semantics=("arbitrary",)),
    )(
        bits, i0, i1, i2, i2,
        g2e['W1'], g2e['b1'], g2e['W2'], g2e['b2'],
        g2d['W1'], g2d['b1'], g2d['W2'], g2d['b2'],
        g0d['W1'], g0d['b1'], g0d['W2'], g0d['b2'],
        g0r['W1'], g0r['b1'], g0r['W2'], g0r['b2'],
        g1e['W1'], g1e['b1'], g1e['W2'], g1e['b2'],
        g1d['W1'], g1d['b1'], g1d['W2'], g1d['b2'],
        g1r['W1'], g1r['b1'], g1r['W2'], g1r['b2'],
    )

    B = inputs_1.shape[0]
    return (out0.reshape(B, D),
            out1.reshape(B, N, D),
            out2.reshape(B, N, N, D))


# speedup vs baseline: 1.2765x; 1.0054x over previous
import jax
import jax.numpy as jnp
from jax import lax
from jax.experimental import pallas as pl
from jax.experimental.pallas import tpu as pltpu

N, D, H = 256, 128, 512
B_ = 64
NI = N // B_
NP = NI * (NI + 1) // 2
R = B_ * B_
F32 = jnp.float32


def _pair(p):
    i = jnp.int32(0)
    start = 0
    for r in range(1, NI):
        start += NI - (r - 1)
        i = i + (p >= start).astype(jnp.int32)
    base = i * NI - (i * (i - 1)) // 2
    j = p - base + i
    return i, j


def _mlp(x, w1, b1, w2, b2):
    h = jnp.maximum(jnp.dot(x, w1, preferred_element_type=F32) + b1, 0.0)
    return jnp.dot(h, w2, preferred_element_type=F32) + b2


def _body(act_ref, i0_ref, i1_ref, x_ref, y_ref,
          w1e_ref, b1e_ref, w2e_ref, b2e_ref,
          w1d_ref, b1d_ref, w2d_ref, b2d_ref,
          w1g0d_ref, b1g0d_ref, w2g0d_ref, b2g0d_ref,
          w1g0r_ref, b1g0r_ref, w2g0r_ref, b2g0r_ref,
          w1g1e_ref, b1g1e_ref, w2g1e_ref, b2g1e_ref,
          w1g1d_ref, b1g1d_ref, w2g1d_ref, b2g1d_ref,
          w1g1r_ref, b1g1r_ref, w2g1r_ref, b2g1r_ref,
          out2_ref, out1_ref, out0_ref,
          accmx_ref, accmn_ref, a_ref, bm_ref, st_ref, sem, w2eb_ref,
          w0x_ref, w0n_ref, w1x_ref, w1n_ref):
    p = pl.program_id(0)
    I, J = _pair(p)
    b = lax.rem(p, 2)
    rows_i = pl.ds(I * B_, B_)
    rows_j = pl.ds(J * B_, B_)

    @pl.when(p == 0)
    def _prep():
        i1v = i1_ref[...]
        a_ref[...] = (jnp.dot(i1v, w1e_ref[0:D, :],
                              preferred_element_type=F32)
                      + b1e_ref[...]).astype(jnp.bfloat16)
        bm_ref[...] = jnp.dot(i1v, w1e_ref[D:, :],
                              preferred_element_type=F32).astype(jnp.bfloat16)
        w2eb_ref[...] = w2e_ref[...].astype(jnp.bfloat16)
        accmx_ref[...] = jnp.full((N, D), -jnp.inf, F32)
        accmn_ref[...] = jnp.full((N, D), jnp.inf, F32)
        ii = lax.broadcasted_iota(jnp.int32, (D, 2 * D), 0)
        jj = lax.broadcasted_iota(jnp.int32, (D, 2 * D), 1)
        se = (jj == 2 * ii).astype(F32)
        so = (jj == 2 * ii + 1).astype(F32)
        w0x_ref[...] = jnp.dot(se, w1g0r_ref[...], preferred_element_type=F32)
        w0n_ref[...] = jnp.dot(so, w1g0r_ref[...], preferred_element_type=F32)
        w1x_ref[...] = jnp.dot(se, w1g1r_ref[...], preferred_element_type=F32)
        w1n_ref[...] = jnp.dot(so, w1g1r_ref[...], preferred_element_type=F32)

    x = x_ref[...]
    y = y_ref[...]

    def bit(k):
        return ((act_ref[0] >> (6 - k)) & 1).astype(F32)

    bits5 = bit(5)
    bits6 = bit(6)

    def exp_term(arows, brows):
        he = jnp.maximum(a_ref[arows, :][:, None, :]
                         + bm_ref[brows, :][None, :, :],
                         jnp.bfloat16(0.0)).reshape(R, H)
        return bits5 * (jnp.dot(he, w2eb_ref[...],
                                preferred_element_type=F32) + b2e_ref[...])

    def dir_term(u, v_t):
        hd = jnp.maximum(
            jnp.dot(u.reshape(R, D), w1d_ref[0:D, :],
                    preferred_element_type=F32)
            + jnp.dot(v_t.reshape(R, D), w1d_ref[D:, :],
                      preferred_element_type=F32)
            + b1d_ref[...], 0.0)
        return bits6 * (jnp.dot(hd, w2d_ref[...], preferred_element_type=F32)
                        + b2d_ref[...])

    z0 = exp_term(rows_i, rows_j)
    st_ref[0, b] = jax.nn.sigmoid(
        z0 + dir_term(x, jnp.swapaxes(y, 0, 1))).reshape(B_, B_, D)
    pltpu.make_async_copy(st_ref.at[0, b],
                          out2_ref.at[rows_i, rows_j, :],
                          sem.at[0, b]).start()

    @pl.when(I != J)
    def _():
        z1 = exp_term(rows_j, rows_i)
        st_ref[1, b] = jax.nn.sigmoid(
            z1 + dir_term(y, jnp.swapaxes(x, 0, 1))).reshape(B_, B_, D)
        pltpu.make_async_copy(st_ref.at[1, b],
                              out2_ref.at[rows_j, rows_i, :],
                              sem.at[1, b]).start()

    accmx_ref[rows_i, :] = jnp.maximum(accmx_ref[rows_i, :], jnp.max(x, 1))
    accmn_ref[rows_i, :] = jnp.minimum(accmn_ref[rows_i, :], jnp.min(x, 1))

    @pl.when(I != J)
    def _():
        accmx_ref[rows_j, :] = jnp.maximum(accmx_ref[rows_j, :],
                                           jnp.max(y, 1))
        accmn_ref[rows_j, :] = jnp.minimum(accmn_ref[rows_j, :],
                                           jnp.min(y, 1))

    @pl.when(p > 0)
    def _():
        q = p - 1
        Iq, Jq = _pair(q)
        bq = lax.rem(q, 2)
        qri = pl.ds(Iq * B_, B_)
        qrj = pl.ds(Jq * B_, B_)
        pltpu.make_async_copy(st_ref.at[0, bq],
                              out2_ref.at[qri, qrj, :],
                              sem.at[0, bq]).wait()

        @pl.when(Iq != Jq)
        def _():
            pltpu.make_async_copy(st_ref.at[1, bq],
                                  out2_ref.at[qrj, qri, :],
                                  sem.at[1, bq]).wait()

    @pl.when(p == NP - 1)
    def _tail():
        pltpu.make_async_copy(st_ref.at[0, b],
                              out2_ref.at[rows_i, rows_j, :],
                              sem.at[0, b]).wait()

        @pl.when(I != J)
        def _():
            pltpu.make_async_copy(st_ref.at[1, b],
                                  out2_ref.at[rows_j, rows_i, :],
                                  sem.at[1, b]).wait()

        i0v = i0_ref[...]
        i1v = i1_ref[...]
        mx1 = jnp.max(i1v, axis=0, keepdims=True)
        mn1 = jnp.min(i1v, axis=0, keepdims=True)
        o0d = _mlp(i0v, w1g0d_ref[...], b1g0d_ref[...],
                   w2g0d_ref[...], b2g0d_ref[...])
        h0r = jnp.maximum(
            jnp.dot(mx1, w0x_ref[...], preferred_element_type=F32)
            + jnp.dot(mn1, w0n_ref[...], preferred_element_type=F32)
            + b1g0r_ref[...], 0.0)
        o0r = jnp.dot(h0r, w2g0r_ref[...], preferred_element_type=F32) \
            + b2g0r_ref[...]
        out0_ref[...] = jax.nn.sigmoid(bit(0) * o0d + bit(1) * o0r)

        e1 = _mlp(i0v, w1g1e_ref[...], b1g1e_ref[...],
                  w2g1e_ref[...], b2g1e_ref[...])
        o1d = _mlp(i1v, w1g1d_ref[...], b1g1d_ref[...],
                   w2g1d_ref[...], b2g1d_ref[...])
        h1r = jnp.maximum(
            jnp.dot(accmx_ref[...], w1x_ref[...], preferred_element_type=F32)
            + jnp.dot(accmn_ref[...], w1n_ref[...],
                      preferred_element_type=F32)
            + b1g1r_ref[...], 0.0)
        o1r = jnp.dot(h1r, w2g1r_ref[...], preferred_element_type=F32) \
            + b2g1r_ref[...]
        out1_ref[...] = jax.nn.sigmoid(
            bit(2) * e1 + bit(3) * o1d + bit(4) * o1r)


def kernel(inputs_0, inputs_1, inputs_2, action, params):
    act = jnp.asarray(action).astype(jnp.int32).reshape(1)

    i0 = inputs_0.reshape(1, D)
    i1 = inputs_1.reshape(N, D)
    i2 = inputs_2.reshape(N, N, D)

    def row(p):
        return {k: (v.reshape(1, -1) if v.ndim == 1 else v)
                for k, v in p.items()}

    g2e = row(params['g2_exp'])
    g2d = row(params['g2_dir'])
    g0d = row(params['g0_dir'])
    g0r = row(params['g0_red'])
    g1e = row(params['g1_exp'])
    g1d = row(params['g1_dir'])
    g1r = row(params['g1_red'])

    const2 = lambda shape: pl.BlockSpec(shape, lambda p: (0, 0))

    def xmap(p):
        i, j = _pair(p)
        return (i, j, 0)

    def ymap(p):
        i, j = _pair(p)
        return (j, i, 0)

    in_specs = [
        pl.BlockSpec(memory_space=pltpu.SMEM),
        const2((1, D)),
        const2((N, D)),
        pl.BlockSpec((B_, B_, D), xmap),
        pl.BlockSpec((B_, B_, D), ymap),
        const2((2 * D, H)), const2((1, H)),
        const2((H, D)), const2((1, D)),
        const2((2 * D, H)), const2((1, H)),
        const2((H, D)), const2((1, D)),
        const2((D, H)), const2((1, H)), const2((H, D)), const2((1, D)),
        const2((2 * D, H)), const2((1, H)),
        const2((H, D)), const2((1, D)),
        const2((D, H)), const2((1, H)), const2((H, D)), const2((1, D)),
        const2((D, H)), const2((1, H)), const2((H, D)), const2((1, D)),
        const2((2 * D, H)), const2((1, H)),
        const2((H, D)), const2((1, D)),
    ]
    out_specs = [
        pl.BlockSpec(memory_space=pl.ANY),
        const2((N, D)),
        const2((1, D)),
    ]
    out_shape = [
        jax.ShapeDtypeStruct((N, N, D), F32),
        jax.ShapeDtypeStruct((N, D), F32),
        jax.ShapeDtypeStruct((1, D), F32),
    ]
    scratch = [
        pltpu.VMEM((N, D), F32),
        pltpu.VMEM((N, D), F32),
        pltpu.VMEM((N, H), jnp.bfloat16),
        pltpu.VMEM((N, H), jnp.bfloat16),
        pltpu.VMEM((2, 2, B_, B_, D), F32),
        pltpu.SemaphoreType.DMA((2, 2)),
        pltpu.VMEM((H, D), jnp.bfloat16),
        pltpu.VMEM((D, H), F32),
        pltpu.VMEM((D, H), F32),
        pltpu.VMEM((D, H), F32),
        pltpu.VMEM((D, H), F32),
    ]

    out2, out1, out0 = pl.pallas_call(
        _body,
        grid=(NP,),
        in_specs=in_specs,
        out_specs=out_specs,
        out_shape=out_shape,
        scratch_shapes=scratch,
        compiler_params=pltpu.CompilerParams(
            dimension_semantics=("arbitrary",)),
    )(
        act, i0, i1, i2, i2,
        g2e['W1'], g2e['b1'], g2e['W2'], g2e['b2'],
        g2d['W1'], g2d['b1'], g2d['W2'], g2d['b2'],
        g0d['W1'], g0d['b1'], g0d['W2'], g0d['b2'],
        g0r['W1'], g0r['b1'], g0r['W2'], g0r['b2'],
        g1e['W1'], g1e['b1'], g1e['W2'], g1e['b2'],
        g1d['W1'], g1d['b1'], g1d['W2'], g1d['b2'],
        g1r['W1'], g1r['b1'], g1r['W2'], g1r['b2'],
    )

    B = inputs_1.shape[0]
    return (out0.reshape(B, D),
            out1.reshape(B, N, D),
            out2.reshape(B, N, N, D))
